# baseline (speedup 1.0000x reference)
import functools

import jax
import jax.numpy as jnp
from jax import lax
from jax.experimental import pallas as pl
from jax.experimental.pallas import tpu as pltpu
from jax.experimental.pallas import tpu_sc as plsc

VOCAB_SIZE = 1000000
HIDDEN = 32
BATCH = 4096
HIST = 200

NUM_CORES = 2
NUM_SUBCORES = 16
NW = NUM_CORES * NUM_SUBCORES
NBG = BATCH // 128
NHG = HIDDEN // 8


def _gather_body(idx_hbm, table_hbm, out_hbm, idx_v, rbuf, obuf, gsems, osems):
    w = lax.axis_index("s") * NUM_CORES + lax.axis_index("c")
    pltpu.sync_copy(idx_hbm.at[:, w], idx_v)

    lane = lax.iota(jnp.int32, 16)

    def start_gather(t, b):
        pltpu.async_copy(table_hbm.at[idx_v.at[t]], rbuf.at[b], gsems[b])

    start_gather(0, 0)
    start_gather(1, 1)

    @pl.loop(0, HIST, step=2)
    def _t2(t0):
        for b in range(2):
            t = t0 + b
            pltpu.make_async_copy(table_hbm.at[idx_v.at[t]], rbuf.at[b], gsems[b]).wait()
            @pl.when(t >= 2)
            def _():
                pltpu.make_async_copy(obuf.at[b], out_hbm.at[t, :, w], osems[b]).wait()
            for hg in range(NHG):
                for hm in range(8):
                    h = hg * 8 + hm
                    col = jnp.full((16,), h, dtype=jnp.int32)
                    for bmb in range(8):
                        rows = bmb * 16 + lane
                        vals = plsc.load_gather(rbuf.at[b], [rows, col])
                        obuf[b, hg, hm, pl.ds(bmb * 16, 16)] = vals
            pltpu.async_copy(obuf.at[b], out_hbm.at[t, :, w], osems[b])

            @pl.when(t + 2 < HIST)
            def _():
                start_gather(t + 2, b)

    for b in range(2):
        pltpu.make_async_copy(obuf.at[b], out_hbm.at[HIST - 2 + b, :, w], osems[b]).wait()


@jax.jit
def _embed(idx3, weight):
    mesh = plsc.VectorSubcoreMesh(core_axis_name="c", subcore_axis_name="s")
    k = functools.partial(
        pl.kernel,
        out_type=jax.ShapeDtypeStruct((HIST, NHG, NBG, 8, 128), jnp.float32),
        mesh=mesh,
        scratch_types=[
            pltpu.VMEM((HIST, 128), jnp.int32),
            pltpu.VMEM((2, 128, HIDDEN), jnp.float32),
            pltpu.VMEM((2, NHG, 8, 128), jnp.float32),
            [pltpu.SemaphoreType.DMA] * 2,
            [pltpu.SemaphoreType.DMA] * 2,
        ],
        compiler_params=pltpu.CompilerParams(
            use_tc_tiling_on_sc=False, needs_layout_passes=False
        ),
    )(_gather_body)
    return k(idx3, weight)


def kernel(x, weight):
    idx3 = jnp.transpose(x).reshape(HIST, NBG, 128).astype(jnp.int32)
    out5 = _embed(idx3, weight)
    return out5.transpose(2, 4, 0, 1, 3).reshape(BATCH, HIST, HIDDEN)


# speedup vs baseline: 1.7552x; 1.7552x over previous
import functools

import jax
import jax.numpy as jnp
from jax import lax
from jax.experimental import pallas as pl
from jax.experimental.pallas import tpu as pltpu
from jax.experimental.pallas import tpu_sc as plsc

VOCAB_SIZE = 1000000
HIDDEN = 32
BATCH = 4096
HIST = 200

NUM_CORES = 2
NUM_SUBCORES = 16
NW = NUM_CORES * NUM_SUBCORES
NBG = BATCH // 128
NHG = HIDDEN // 8


def _gather_body(idx_hbm, table_hbm, out_hbm, idx_v, rbuf, obuf, gsems, osems):
    w = lax.axis_index("s") * NUM_CORES + lax.axis_index("c")
    pltpu.sync_copy(idx_hbm.at[:, w], idx_v)

    lane = lax.iota(jnp.int32, 16)
    hm_idx = lane & 7
    hg_half = lane >> 3

    def start_gather(t, b):
        pltpu.async_copy(table_hbm.at[idx_v.at[t]], rbuf.at[b], gsems[b])

    start_gather(0, 0)
    start_gather(1, 1)

    def out_slice(t):
        return out_hbm.at[t, :, w]

    def obuf_slice(b):
        return obuf.at[b, :, :, pl.ds(0, 128)]

    @pl.loop(0, HIST, step=2)
    def _t2(t0):
        for b in range(2):
            t = t0 + b
            pltpu.make_async_copy(table_hbm.at[idx_v.at[t]], rbuf.at[b], gsems[b]).wait()
            @pl.when(t >= 2)
            def _():
                pltpu.make_async_copy(obuf_slice(b), out_slice(t), osems[b]).wait()

            @pl.loop(0, 128, step=8)
            def _bm8(bm0):
                for db in range(8):
                    bm = bm0 + db
                    bm_vec = jnp.full((16,), 0, jnp.int32) + bm
                    for hh in range(2):
                        vals = rbuf[b, bm, pl.ds(hh * 16, 16)]
                        plsc.store_scatter(
                            obuf.at[b], [hg_half + 2 * hh, hm_idx, bm_vec], vals
                        )

            pltpu.async_copy(obuf_slice(b), out_slice(t), osems[b])

            @pl.when(t + 2 < HIST)
            def _():
                start_gather(t + 2, b)

    for b in range(2):
        pltpu.make_async_copy(obuf_slice(b), out_slice(HIST - 2 + b), osems[b]).wait()


@jax.jit
def _embed(idx3, weight):
    mesh = plsc.VectorSubcoreMesh(core_axis_name="c", subcore_axis_name="s")
    k = functools.partial(
        pl.kernel,
        out_type=jax.ShapeDtypeStruct((HIST, NHG, NBG, 8, 128), jnp.float32),
        mesh=mesh,
        scratch_types=[
            pltpu.VMEM((HIST, 128), jnp.int32),
            pltpu.VMEM((2, 128, HIDDEN), jnp.float32),
            pltpu.VMEM((2, NHG, 8, 129), jnp.float32),
            [pltpu.SemaphoreType.DMA] * 2,
            [pltpu.SemaphoreType.DMA] * 2,
        ],
        compiler_params=pltpu.CompilerParams(
            use_tc_tiling_on_sc=False, needs_layout_passes=False
        ),
    )(_gather_body)
    return k(idx3, weight)


def kernel(x, weight):
    idx3 = jnp.transpose(x).reshape(HIST, NBG, 128).astype(jnp.int32)
    out5 = _embed(idx3, weight)
    return out5.transpose(2, 4, 0, 1, 3).reshape(BATCH, HIST, HIDDEN)
